# baseline (speedup 1.0000x reference)
import functools

import jax
import jax.numpy as jnp
from jax import lax
from jax.experimental import pallas as pl
from jax.experimental.pallas import tpu as pltpu
from jax.experimental.pallas import tpu_sc as plsc

EPS_LUB = 0.0001

_GATHER_DNUMS = lax.GatherDimensionNumbers(
    offset_dims=(), collapsed_slice_dims=(0,), start_index_map=(0,)
)


def _permute16(v, idx):
    return lax.gather(
        v,
        idx[:, None],
        _GATHER_DNUMS,
        slice_sizes=(1,),
        mode=lax.GatherScatterMode.PROMISE_IN_BOUNDS,
    )


N_NODES = 100000
N_EDGES = 6400000
NUM_PAIRS = 16
CHUNK = 4000
HALF = CHUNK // 2
EDGES_PER_PAIR = N_EDGES // NUM_PAIRS
NUM_CHUNKS = EDGES_PER_PAIR // CHUNK
PREF = 8000
NUM_PTILES = 2 * N_NODES // PREF


def _body(fcols_hbm, fvflat_hbm, eflat_hbm, out_hbm,
          tbl, ebuf, cbuf, vxbuf, vybuf, ibuf, shared):
    c = lax.axis_index("c")
    s = lax.axis_index("s")
    col = s & 1
    sp = s >> 1
    pair = c * 8 + sp
    wid = c * 16 + s

    pltpu.sync_copy(fcols_hbm.at[col], tbl)

    @pl.when(wid < NUM_PTILES)
    def _copy_prefix():
        pltpu.sync_copy(fvflat_hbm.at[pl.ds(wid * PREF, CHUNK)], ibuf)
        pltpu.sync_copy(ibuf, out_hbm.at[pl.ds(wid * PREF, CHUNK)])
        pltpu.sync_copy(
            fvflat_hbm.at[pl.ds(wid * PREF + CHUNK, CHUNK)], ibuf
        )
        pltpu.sync_copy(ibuf, out_hbm.at[pl.ds(wid * PREF + CHUNK, CHUNK)])

    lanes = lax.iota(jnp.int32, 16)
    swap = lanes ^ 1
    pat = lanes >> 1

    pair_base = pair * EDGES_PER_PAIR

    def chunk_body(k, carry):
        ebase = pair_base + k * CHUNK
        slot = k & 1
        pltpu.sync_copy(eflat_hbm.at[pl.ds(ebase * 2, 2 * CHUNK)], ebuf)

        def gather_loop(j, carry2):
            va = ebuf[pl.ds(j * 32, 16)]
            vb = ebuf[pl.ds(j * 32 + 16, 16)]
            ga = plsc.load_gather(tbl, [va])
            gb = plsc.load_gather(tbl, [vb])
            sa = _permute16(ga, swap)
            sb = _permute16(gb, swap)
            ma = jnp.maximum(ga, sa) + EPS_LUB
            mb = jnp.maximum(gb, sb) + EPS_LUB
            base_o = j * 16
            plsc.store_scatter(cbuf, [base_o + pat], ma)
            plsc.store_scatter(cbuf, [base_o + 8 + pat], mb)
            return carry2

        lax.fori_loop(0, CHUNK // 16, gather_loop, 0, unroll=4)

        pltpu.sync_copy(cbuf, shared.at[sp, col, slot])
        plsc.subcore_barrier()

        h0 = col * HALF
        pltpu.sync_copy(shared.at[sp, 0, slot, pl.ds(h0, HALF)], vxbuf)
        pltpu.sync_copy(shared.at[sp, 1, slot, pl.ds(h0, HALF)], vybuf)

        def ilv_loop(j, carry2):
            vx = vxbuf[pl.ds(j * 16, 16)]
            vy = vybuf[pl.ds(j * 16, 16)]
            pos = (j * 32) + 2 * lanes
            plsc.store_scatter(ibuf, [pos], vx)
            plsc.store_scatter(ibuf, [pos + 1], vy)
            return carry2

        lax.fori_loop(0, HALF // 16, ilv_loop, 0, unroll=4)
        pltpu.sync_copy(
            ibuf,
            out_hbm.at[pl.ds(2 * (N_NODES + ebase) + col * CHUNK, CHUNK)],
        )
        return carry

    lax.fori_loop(0, NUM_CHUNKS, chunk_body, 0)


@jax.jit
def _run(fcols, fvflat, eflat):
    mesh = plsc.VectorSubcoreMesh(
        core_axis_name="c", subcore_axis_name="s"
    )
    k = functools.partial(
        pl.kernel,
        mesh=mesh,
        out_type=jax.ShapeDtypeStruct((2 * (N_NODES + N_EDGES),), jnp.float32),
        scratch_types=[
            pltpu.VMEM((N_NODES,), jnp.float32),
            pltpu.VMEM((2 * CHUNK,), jnp.int32),
            pltpu.VMEM((CHUNK,), jnp.float32),
            pltpu.VMEM((HALF,), jnp.float32),
            pltpu.VMEM((HALF,), jnp.float32),
            pltpu.VMEM((CHUNK,), jnp.float32),
            pltpu.VMEM_SHARED((8, 2, 2, CHUNK), jnp.float32),
        ],
        compiler_params=pltpu.CompilerParams(
            use_tc_tiling_on_sc=False, needs_layout_passes=False
        ),
    )(_body)
    return k(fcols, fvflat, eflat)


def kernel(f_v, edges):
    f_v = f_v.astype(jnp.float32)
    fcols = f_v.T
    fvflat = f_v.reshape(-1)
    eflat = edges.astype(jnp.int32).reshape(-1)
    out_flat = _run(fcols, fvflat, eflat)
    return out_flat.reshape(N_NODES + N_EDGES, 2)


# speedup vs baseline: 25.2889x; 25.2889x over previous
import functools

import jax
import jax.numpy as jnp
from jax import lax
from jax.experimental import pallas as pl
from jax.experimental.pallas import tpu as pltpu
from jax.experimental.pallas import tpu_sc as plsc

EPS_LUB = 0.0001

N_NODES = 100000
N_EDGES = 6400000
M_ROWS = N_NODES + N_EDGES
NEB = N_EDGES // 128
NOB = (M_ROWS + 127) // 128
NVB = N_NODES // 128
VREM = N_NODES - NVB * 128
BOUND = NVB
FIRST_EB = BOUND + 1
LAST_EB = NOB - 1
MAIN_BLOCKS = LAST_EB - FIRST_EB
BPT = 3125
OB = 25
KOFF = [(96 + 16 * g) // 128 for g in range(8)]
OOFF = [(96 + 16 * g) % 128 for g in range(8)]


def _emit_block(tbl, ebuf, obuf, t):
    for g in range(8):
        v0 = ebuf[t + KOFF[g], 0, pl.ds(OOFF[g], 16)]
        v1 = ebuf[t + KOFF[g], 1, pl.ds(OOFF[g], 16)]
        g0 = plsc.load_gather(tbl, [v0])
        g1 = plsc.load_gather(tbl, [v1])
        obuf[t, 0, pl.ds(16 * g, 16)] = jnp.maximum(g0, g1) + EPS_LUB


def _body(fcols_hbm, eblk_hbm, out_hbm, tbl, ebuf, obuf, bebuf, bbuf):
    c = lax.axis_index("c")
    s = lax.axis_index("s")
    col = c

    pltpu.sync_copy(fcols_hbm.at[col], tbl)

    def vblock(k, carry):
        b = s + 16 * k

        @pl.when(b < NVB)
        def _():
            pltpu.sync_copy(tbl.at[pl.ds(b * 128, 128)], out_hbm.at[b, col])

        return carry

    lax.fori_loop(0, (NVB + 15) // 16, vblock, 0)

    @pl.when(s == 0)
    def _boundary():
        pltpu.sync_copy(eblk_hbm.at[0], bebuf)
        for g in range(2):
            bbuf[pl.ds(16 * g, 16)] = tbl[pl.ds(NVB * 128 + 16 * g, 16)]
        for g in range(2, 8):
            v0 = bebuf[0, pl.ds(16 * g - 32, 16)]
            v1 = bebuf[1, pl.ds(16 * g - 32, 16)]
            g0 = plsc.load_gather(tbl, [v0])
            g1 = plsc.load_gather(tbl, [v1])
            bbuf[pl.ds(16 * g, 16)] = jnp.maximum(g0, g1) + EPS_LUB
        pltpu.sync_copy(bbuf, out_hbm.at[BOUND, col])

    @pl.when(s == 1)
    def _tail():
        pltpu.sync_copy(eblk_hbm.at[NEB - 1], bebuf)
        for g in range(2):
            v0 = bebuf[0, pl.ds(96 + 16 * g, 16)]
            v1 = bebuf[1, pl.ds(96 + 16 * g, 16)]
            g0 = plsc.load_gather(tbl, [v0])
            g1 = plsc.load_gather(tbl, [v1])
            m = jnp.maximum(g0, g1) + EPS_LUB
            for g2 in range(g, 8, 2):
                bbuf[pl.ds(16 * g2, 16)] = m
        pltpu.sync_copy(bbuf, out_hbm.at[LAST_EB, col])

    tile_b0 = FIRST_EB + s * BPT

    def chunk(k, carry, nob):
        ob0 = tile_b0 + k * OB
        ib0 = ob0 - FIRST_EB
        pltpu.sync_copy(eblk_hbm.at[pl.ds(ib0, nob + 1)],
                        ebuf.at[pl.ds(0, nob + 1)])

        def blk(t, carry2):
            _emit_block(tbl, ebuf, obuf, t)
            return carry2

        lax.fori_loop(0, nob, blk, 0)
        pltpu.sync_copy(
            obuf.at[pl.ds(0, nob)],
            out_hbm.at[pl.ds(ob0, nob), pl.ds(col, 1), :],
        )
        return carry

    lax.fori_loop(0, BPT // OB - jnp.where(s == 15, 1, 0),
                  functools.partial(chunk, nob=OB), 0)

    @pl.when(s == 15)
    def _short_tail():
        chunk((BPT // OB) - 1, 0, nob=OB - 1)


@jax.jit
def _run(fcols, eblk):
    mesh = plsc.VectorSubcoreMesh(
        core_axis_name="c", subcore_axis_name="s"
    )
    k = functools.partial(
        pl.kernel,
        mesh=mesh,
        out_type=jax.ShapeDtypeStruct((NOB, 2, 128), jnp.float32),
        scratch_types=[
            pltpu.VMEM((N_NODES,), jnp.float32),
            pltpu.VMEM((OB + 1, 2, 128), jnp.int32),
            pltpu.VMEM((OB, 1, 128), jnp.float32),
            pltpu.VMEM((2, 128), jnp.int32),
            pltpu.VMEM((128,), jnp.float32),
        ],
        compiler_params=pltpu.CompilerParams(
            use_tc_tiling_on_sc=False, needs_layout_passes=False
        ),
    )(_body)
    return k(fcols, eblk)


def kernel(f_v, edges):
    f_v = f_v.astype(jnp.float32)
    fcols = f_v.T
    eblk = (
        edges.astype(jnp.int32)
        .reshape(NEB, 128, 2)
        .transpose(0, 2, 1)
    )
    out_blk = _run(fcols, eblk)
    out = out_blk.transpose(0, 2, 1).reshape(NOB * 128, 2)
    return out[:M_ROWS]


# speedup vs baseline: 34.2384x; 1.3539x over previous
import functools

import jax
import jax.numpy as jnp
from jax import lax
from jax.experimental import pallas as pl
from jax.experimental.pallas import tpu as pltpu
from jax.experimental.pallas import tpu_sc as plsc

EPS_LUB = 0.0001

N_NODES = 100000
N_EDGES = 6400000
M_ROWS = N_NODES + N_EDGES
NEB = N_EDGES // 128
NOB = (M_ROWS + 127) // 128
NVB = N_NODES // 128
VREM = N_NODES - NVB * 128
BOUND = NVB
FIRST_EB = BOUND + 1
LAST_EB = NOB - 1
MAIN_BLOCKS = LAST_EB - FIRST_EB
BPT = 3125
OB = 25
KOFF = [(96 + 16 * g) // 128 for g in range(8)]
OOFF = [(96 + 16 * g) % 128 for g in range(8)]


def _emit_block(tbl, ebuf, obuf, t):
    for g in range(8):
        v0 = ebuf[t + KOFF[g], 0, pl.ds(OOFF[g], 16)]
        v1 = ebuf[t + KOFF[g], 1, pl.ds(OOFF[g], 16)]
        g0 = plsc.load_gather(tbl, [v0])
        g1 = plsc.load_gather(tbl, [v1])
        obuf[t, 0, pl.ds(16 * g, 16)] = jnp.maximum(g0, g1) + EPS_LUB


def _body(fcols_hbm, eblk_hbm, out_hbm, tbl, ebuf, obuf, bebuf, bbuf,
          semi, semo):
    c = lax.axis_index("c")
    s = lax.axis_index("s")
    col = c

    pltpu.sync_copy(fcols_hbm.at[col], tbl)

    def vblock(k, carry):
        b = s + 16 * k

        @pl.when(b < NVB)
        def _():
            pltpu.sync_copy(tbl.at[pl.ds(b * 128, 128)], out_hbm.at[b, col])

        return carry

    lax.fori_loop(0, (NVB + 15) // 16, vblock, 0)

    @pl.when(s == 0)
    def _boundary():
        pltpu.sync_copy(eblk_hbm.at[0], bebuf)
        for g in range(2):
            bbuf[pl.ds(16 * g, 16)] = tbl[pl.ds(NVB * 128 + 16 * g, 16)]
        for g in range(2, 8):
            v0 = bebuf[0, pl.ds(16 * g - 32, 16)]
            v1 = bebuf[1, pl.ds(16 * g - 32, 16)]
            g0 = plsc.load_gather(tbl, [v0])
            g1 = plsc.load_gather(tbl, [v1])
            bbuf[pl.ds(16 * g, 16)] = jnp.maximum(g0, g1) + EPS_LUB
        pltpu.sync_copy(bbuf, out_hbm.at[BOUND, col])

    @pl.when(s == 1)
    def _tail():
        pltpu.sync_copy(eblk_hbm.at[NEB - 1], bebuf)
        for g in range(2):
            v0 = bebuf[0, pl.ds(96 + 16 * g, 16)]
            v1 = bebuf[1, pl.ds(96 + 16 * g, 16)]
            g0 = plsc.load_gather(tbl, [v0])
            g1 = plsc.load_gather(tbl, [v1])
            m = jnp.maximum(g0, g1) + EPS_LUB
            for g2 in range(g, 8, 2):
                bbuf[pl.ds(16 * g2, 16)] = m
        pltpu.sync_copy(bbuf, out_hbm.at[LAST_EB, col])

    tile_b0 = FIRST_EB + s * BPT
    nch = BPT // OB - jnp.where(s == 15, 1, 0)

    def in_copy(k, slot):
        ib0 = tile_b0 - FIRST_EB + k * OB
        return pltpu.make_async_copy(
            eblk_hbm.at[pl.ds(ib0, OB + 1)], ebuf.at[slot], semi.at[slot]
        )

    def out_copy(k, slot):
        ob0 = tile_b0 + k * OB
        return pltpu.make_async_copy(
            obuf.at[slot],
            out_hbm.at[pl.ds(ob0, OB), pl.ds(col, 1), :],
            semo.at[slot],
        )

    in_copy(0, 0).start()

    def chunk(k, carry):
        slot = k & 1
        in_copy(k, slot).wait()

        @pl.when(k + 1 < nch)
        def _():
            in_copy(k + 1, 1 - slot).start()

        @pl.when(k >= 2)
        def _():
            out_copy(k - 2, slot).wait()

        def blk(t, carry2):
            _emit_block(tbl, ebuf.at[slot], obuf.at[slot], t)
            return carry2

        lax.fori_loop(0, OB, blk, 0)
        out_copy(k, slot).start()
        return carry

    lax.fori_loop(0, nch, chunk, 0)
    out_copy(nch - 2, nch & 1).wait()
    out_copy(nch - 1, 1 - (nch & 1)).wait()

    @pl.when(s == 15)
    def _short_tail():
        k = BPT // OB - 1
        ob0 = tile_b0 + k * OB
        ib0 = ob0 - FIRST_EB
        pltpu.sync_copy(eblk_hbm.at[pl.ds(ib0, OB)],
                        ebuf.at[0, pl.ds(0, OB)])

        def blk(t, carry2):
            _emit_block(tbl, ebuf.at[0], obuf.at[0], t)
            return carry2

        lax.fori_loop(0, OB - 1, blk, 0)
        pltpu.sync_copy(
            obuf.at[0, pl.ds(0, OB - 1)],
            out_hbm.at[pl.ds(ob0, OB - 1), pl.ds(col, 1), :],
        )


@jax.jit
def _run(fcols, eblk):
    mesh = plsc.VectorSubcoreMesh(
        core_axis_name="c", subcore_axis_name="s"
    )
    k = functools.partial(
        pl.kernel,
        mesh=mesh,
        out_type=jax.ShapeDtypeStruct((NOB, 2, 128), jnp.float32),
        scratch_types=[
            pltpu.VMEM((N_NODES,), jnp.float32),
            pltpu.VMEM((2, OB + 1, 2, 128), jnp.int32),
            pltpu.VMEM((2, OB, 1, 128), jnp.float32),
            pltpu.VMEM((2, 128), jnp.int32),
            pltpu.VMEM((128,), jnp.float32),
            pltpu.SemaphoreType.DMA((2,)),
            pltpu.SemaphoreType.DMA((2,)),
        ],
        compiler_params=pltpu.CompilerParams(
            use_tc_tiling_on_sc=False, needs_layout_passes=False
        ),
    )(_body)
    return k(fcols, eblk)


def kernel(f_v, edges):
    f_v = f_v.astype(jnp.float32)
    fcols = f_v.T
    eblk = (
        edges.astype(jnp.int32)
        .reshape(NEB, 128, 2)
        .transpose(0, 2, 1)
    )
    out_blk = _run(fcols, eblk)
    out = out_blk.transpose(0, 2, 1).reshape(NOB * 128, 2)
    return out[:M_ROWS]


# speedup vs baseline: 59.7740x; 1.7458x over previous
import functools

import jax
import jax.numpy as jnp
from jax import lax
from jax.experimental import pallas as pl
from jax.experimental.pallas import tpu as pltpu
from jax.experimental.pallas import tpu_sc as plsc

EPS_LUB = 0.0001

N_NODES = 100000
N_EDGES = 6400000
M_ROWS = N_NODES + N_EDGES
NEB = N_EDGES // 128
NOB = (M_ROWS + 127) // 128
NVB = N_NODES // 128
VREM = N_NODES - NVB * 128
BOUND = NVB
FIRST_EB = BOUND + 1
LAST_EB = NOB - 1
MAIN_BLOCKS = LAST_EB - FIRST_EB
BPT = 3125
OB = 25
KOFF = [(96 + 16 * g) // 128 for g in range(8)]
OOFF = [(96 + 16 * g) % 128 for g in range(8)]


def _emit_block(tbl, ebuf, obuf, t):
    for g in range(8):
        v0 = ebuf[t + KOFF[g], 0, pl.ds(OOFF[g], 16)]
        v1 = ebuf[t + KOFF[g], 1, pl.ds(OOFF[g], 16)]
        g0 = plsc.load_gather(tbl, [v0])
        g1 = plsc.load_gather(tbl, [v1])
        obuf[t, 0, pl.ds(16 * g, 16)] = jnp.maximum(g0, g1) + EPS_LUB


def _body(fcols_hbm, eblk_hbm, out_hbm, tbl, ebuf, obuf, bebuf, bbuf,
          semi, semo):
    c = lax.axis_index("c")
    s = lax.axis_index("s")
    col = c

    pltpu.sync_copy(fcols_hbm.at[col], tbl)

    def vblock(k, carry):
        b = s + 16 * k

        @pl.when(b < NVB)
        def _():
            pltpu.sync_copy(tbl.at[pl.ds(b * 128, 128)], out_hbm.at[b, col])

        return carry

    lax.fori_loop(0, (NVB + 15) // 16, vblock, 0)

    @pl.when(s == 0)
    def _boundary():
        pltpu.sync_copy(eblk_hbm.at[0], bebuf)
        for g in range(2):
            bbuf[pl.ds(16 * g, 16)] = tbl[pl.ds(NVB * 128 + 16 * g, 16)]
        for g in range(2, 8):
            v0 = bebuf[0, pl.ds(16 * g - 32, 16)]
            v1 = bebuf[1, pl.ds(16 * g - 32, 16)]
            g0 = plsc.load_gather(tbl, [v0])
            g1 = plsc.load_gather(tbl, [v1])
            bbuf[pl.ds(16 * g, 16)] = jnp.maximum(g0, g1) + EPS_LUB
        pltpu.sync_copy(bbuf, out_hbm.at[BOUND, col])

    @pl.when(s == 1)
    def _tail():
        pltpu.sync_copy(eblk_hbm.at[NEB - 1], bebuf)
        for g in range(2):
            v0 = bebuf[0, pl.ds(96 + 16 * g, 16)]
            v1 = bebuf[1, pl.ds(96 + 16 * g, 16)]
            g0 = plsc.load_gather(tbl, [v0])
            g1 = plsc.load_gather(tbl, [v1])
            m = jnp.maximum(g0, g1) + EPS_LUB
            for g2 in range(g, 8, 2):
                bbuf[pl.ds(16 * g2, 16)] = m
        pltpu.sync_copy(bbuf, out_hbm.at[LAST_EB, col])

    tile_b0 = FIRST_EB + s * BPT
    nch = BPT // OB - jnp.where(s == 15, 1, 0)

    def in_copy(k, slot):
        ib0 = tile_b0 - FIRST_EB + k * OB
        return pltpu.make_async_copy(
            eblk_hbm.at[pl.ds(ib0, OB + 1)], ebuf.at[slot], semi.at[slot]
        )

    def out_copy(k, slot):
        ob0 = tile_b0 + k * OB
        return pltpu.make_async_copy(
            obuf.at[slot],
            out_hbm.at[pl.ds(ob0, OB), pl.ds(col, 1), :],
            semo.at[slot],
        )

    in_copy(0, 0).start()

    def chunk(k, carry):
        slot = k & 1
        in_copy(k, slot).wait()

        @pl.when(k + 1 < nch)
        def _():
            in_copy(k + 1, 1 - slot).start()

        @pl.when(k >= 2)
        def _():
            out_copy(k - 2, slot).wait()

        @plsc.parallel_loop(0, OB, unroll=2)
        def _blocks(t):
            _emit_block(tbl, ebuf.at[slot], obuf.at[slot], t)

        out_copy(k, slot).start()
        return carry

    lax.fori_loop(0, nch, chunk, 0)
    out_copy(nch - 2, nch & 1).wait()
    out_copy(nch - 1, 1 - (nch & 1)).wait()

    @pl.when(s == 15)
    def _short_tail():
        k = BPT // OB - 1
        ob0 = tile_b0 + k * OB
        ib0 = ob0 - FIRST_EB
        pltpu.sync_copy(eblk_hbm.at[pl.ds(ib0, OB)],
                        ebuf.at[0, pl.ds(0, OB)])

        @plsc.parallel_loop(0, OB - 1, unroll=2)
        def _blocks(t):
            _emit_block(tbl, ebuf.at[0], obuf.at[0], t)
        pltpu.sync_copy(
            obuf.at[0, pl.ds(0, OB - 1)],
            out_hbm.at[pl.ds(ob0, OB - 1), pl.ds(col, 1), :],
        )


@jax.jit
def _run(fcols, eblk):
    mesh = plsc.VectorSubcoreMesh(
        core_axis_name="c", subcore_axis_name="s"
    )
    k = functools.partial(
        pl.kernel,
        mesh=mesh,
        out_type=jax.ShapeDtypeStruct((NOB, 2, 128), jnp.float32),
        scratch_types=[
            pltpu.VMEM((N_NODES,), jnp.float32),
            pltpu.VMEM((2, OB + 1, 2, 128), jnp.int32),
            pltpu.VMEM((2, OB, 1, 128), jnp.float32),
            pltpu.VMEM((2, 128), jnp.int32),
            pltpu.VMEM((128,), jnp.float32),
            pltpu.SemaphoreType.DMA((2,)),
            pltpu.SemaphoreType.DMA((2,)),
        ],
        compiler_params=pltpu.CompilerParams(
            use_tc_tiling_on_sc=False, needs_layout_passes=False
        ),
    )(_body)
    return k(fcols, eblk)


def kernel(f_v, edges):
    f_v = f_v.astype(jnp.float32)
    fcols = f_v.T
    eblk = (
        edges.astype(jnp.int32)
        .reshape(NEB, 128, 2)
        .transpose(0, 2, 1)
    )
    out_blk = _run(fcols, eblk)
    out = out_blk.transpose(0, 2, 1).reshape(NOB * 128, 2)
    return out[:M_ROWS]


# speedup vs baseline: 60.5439x; 1.0129x over previous
import functools

import jax
import jax.numpy as jnp
from jax import lax
from jax.experimental import pallas as pl
from jax.experimental.pallas import tpu as pltpu
from jax.experimental.pallas import tpu_sc as plsc

EPS_LUB = 0.0001

N_NODES = 100000
N_EDGES = 6400000
M_ROWS = N_NODES + N_EDGES
NEB = N_EDGES // 128
NOB = (M_ROWS + 127) // 128
NVB = N_NODES // 128
VREM = N_NODES - NVB * 128
BOUND = NVB
FIRST_EB = BOUND + 1
LAST_EB = NOB - 1
MAIN_BLOCKS = LAST_EB - FIRST_EB
BPT = 3125
OB = 25
KOFF = [(96 + 16 * g) // 128 for g in range(8)]
OOFF = [(96 + 16 * g) % 128 for g in range(8)]


def _emit_block(tbl, ebuf, obuf, t):
    for g in range(8):
        v0 = ebuf[t + KOFF[g], 0, pl.ds(OOFF[g], 16)]
        v1 = ebuf[t + KOFF[g], 1, pl.ds(OOFF[g], 16)]
        g0 = plsc.load_gather(tbl, [v0])
        g1 = plsc.load_gather(tbl, [v1])
        obuf[t, 0, pl.ds(16 * g, 16)] = jnp.maximum(g0, g1) + EPS_LUB


def _body(fcols_hbm, eblk_hbm, out_hbm, tbl, ebuf, obuf, bebuf, bbuf,
          semi, semo, vsem):
    c = lax.axis_index("c")
    s = lax.axis_index("s")
    col = c

    pltpu.sync_copy(fcols_hbm.at[col], tbl)

    def vcopy(b):
        return pltpu.make_async_copy(
            tbl.at[pl.ds(b * 128, 128)], out_hbm.at[b, col], vsem
        )

    def vblock(k, carry):
        b = s + 16 * k

        @pl.when(b < NVB)
        def _():
            vcopy(b).start()

        return carry

    lax.fori_loop(0, (NVB + 15) // 16, vblock, 0)

    @pl.when(s == 0)
    def _boundary():
        pltpu.sync_copy(eblk_hbm.at[0], bebuf)
        for g in range(2):
            bbuf[pl.ds(16 * g, 16)] = tbl[pl.ds(NVB * 128 + 16 * g, 16)]
        for g in range(2, 8):
            v0 = bebuf[0, pl.ds(16 * g - 32, 16)]
            v1 = bebuf[1, pl.ds(16 * g - 32, 16)]
            g0 = plsc.load_gather(tbl, [v0])
            g1 = plsc.load_gather(tbl, [v1])
            bbuf[pl.ds(16 * g, 16)] = jnp.maximum(g0, g1) + EPS_LUB
        pltpu.sync_copy(bbuf, out_hbm.at[BOUND, col])

    @pl.when(s == 1)
    def _tail():
        pltpu.sync_copy(eblk_hbm.at[NEB - 1], bebuf)
        for g in range(2):
            v0 = bebuf[0, pl.ds(96 + 16 * g, 16)]
            v1 = bebuf[1, pl.ds(96 + 16 * g, 16)]
            g0 = plsc.load_gather(tbl, [v0])
            g1 = plsc.load_gather(tbl, [v1])
            m = jnp.maximum(g0, g1) + EPS_LUB
            for g2 in range(g, 8, 2):
                bbuf[pl.ds(16 * g2, 16)] = m
        pltpu.sync_copy(bbuf, out_hbm.at[LAST_EB, col])

    tile_b0 = FIRST_EB + s * BPT
    nch = BPT // OB - jnp.where(s == 15, 1, 0)

    def in_copy(k, slot):
        ib0 = tile_b0 - FIRST_EB + k * OB
        return pltpu.make_async_copy(
            eblk_hbm.at[pl.ds(ib0, OB + 1)], ebuf.at[slot], semi.at[slot]
        )

    def out_copy(k, slot):
        ob0 = tile_b0 + k * OB
        return pltpu.make_async_copy(
            obuf.at[slot],
            out_hbm.at[pl.ds(ob0, OB), pl.ds(col, 1), :],
            semo.at[slot],
        )

    in_copy(0, 0).start()

    def chunk(k, carry):
        slot = k & 1
        in_copy(k, slot).wait()

        @pl.when(k + 1 < nch)
        def _():
            in_copy(k + 1, 1 - slot).start()

        @pl.when(k >= 2)
        def _():
            out_copy(k - 2, slot).wait()

        @plsc.parallel_loop(0, OB, unroll=4)
        def _blocks(t):
            _emit_block(tbl, ebuf.at[slot], obuf.at[slot], t)

        out_copy(k, slot).start()
        return carry

    lax.fori_loop(0, nch, chunk, 0)
    out_copy(nch - 2, nch & 1).wait()
    out_copy(nch - 1, 1 - (nch & 1)).wait()

    def vdrain(k, carry):
        b = s + 16 * k

        @pl.when(b < NVB)
        def _():
            vcopy(b).wait()

        return carry

    lax.fori_loop(0, (NVB + 15) // 16, vdrain, 0)

    @pl.when(s == 15)
    def _short_tail():
        k = BPT // OB - 1
        ob0 = tile_b0 + k * OB
        ib0 = ob0 - FIRST_EB
        pltpu.sync_copy(eblk_hbm.at[pl.ds(ib0, OB)],
                        ebuf.at[0, pl.ds(0, OB)])

        @plsc.parallel_loop(0, OB - 1, unroll=2)
        def _blocks(t):
            _emit_block(tbl, ebuf.at[0], obuf.at[0], t)
        pltpu.sync_copy(
            obuf.at[0, pl.ds(0, OB - 1)],
            out_hbm.at[pl.ds(ob0, OB - 1), pl.ds(col, 1), :],
        )


@jax.jit
def _run(fcols, eblk):
    mesh = plsc.VectorSubcoreMesh(
        core_axis_name="c", subcore_axis_name="s"
    )
    k = functools.partial(
        pl.kernel,
        mesh=mesh,
        out_type=jax.ShapeDtypeStruct((NOB, 2, 128), jnp.float32),
        scratch_types=[
            pltpu.VMEM((N_NODES,), jnp.float32),
            pltpu.VMEM((2, OB + 1, 2, 128), jnp.int32),
            pltpu.VMEM((2, OB, 1, 128), jnp.float32),
            pltpu.VMEM((2, 128), jnp.int32),
            pltpu.VMEM((128,), jnp.float32),
            pltpu.SemaphoreType.DMA((2,)),
            pltpu.SemaphoreType.DMA((2,)),
            pltpu.SemaphoreType.DMA,
        ],
        compiler_params=pltpu.CompilerParams(
            use_tc_tiling_on_sc=False, needs_layout_passes=False
        ),
    )(_body)
    return k(fcols, eblk)


def kernel(f_v, edges):
    f_v = f_v.astype(jnp.float32)
    fcols = f_v.T
    eblk = (
        edges.astype(jnp.int32)
        .reshape(NEB, 128, 2)
        .transpose(0, 2, 1)
    )
    out_blk = _run(fcols, eblk)
    out = out_blk.transpose(0, 2, 1).reshape(NOB * 128, 2)
    return out[:M_ROWS]


# speedup vs baseline: 75.0930x; 1.2403x over previous
import functools

import jax
import jax.numpy as jnp
from jax import lax
from jax.experimental import pallas as pl
from jax.experimental.pallas import tpu as pltpu
from jax.experimental.pallas import tpu_sc as plsc

EPS_LUB = 0.0001

N_NODES = 100000
N_EDGES = 6400000
M_ROWS = N_NODES + N_EDGES
NEB = N_EDGES // 128
NOB = (M_ROWS + 127) // 128
NVB = N_NODES // 128
VREM = N_NODES - NVB * 128
BOUND = NVB
FIRST_EB = BOUND + 1
LAST_EB = NOB - 1
MAIN_BLOCKS = LAST_EB - FIRST_EB
BPT = 3125
OB = 25
KOFF = [(96 + 16 * g) // 128 for g in range(8)]
OOFF = [(96 + 16 * g) % 128 for g in range(8)]


def _emit_block(tbl, ebuf, obuf, t):
    for g in range(8):
        v0 = ebuf[t + KOFF[g], 0, pl.ds(OOFF[g], 16)]
        v1 = ebuf[t + KOFF[g], 1, pl.ds(OOFF[g], 16)]
        g0 = plsc.load_gather(tbl, [v0])
        g1 = plsc.load_gather(tbl, [v1])
        obuf[t, 0, pl.ds(16 * g, 16)] = jnp.maximum(g0, g1) + EPS_LUB


def _body(fcols_hbm, eblk_hbm, out_hbm, tbl, ebuf, obuf, bebuf, bbuf,
          semi, semo, vsem):
    c = lax.axis_index("c")
    s = lax.axis_index("s")
    col = c

    pltpu.sync_copy(fcols_hbm.at[col], tbl)

    def vcopy(b):
        return pltpu.make_async_copy(
            tbl.at[pl.ds(b * 128, 128)], out_hbm.at[b, col], vsem
        )

    def vblock(k, carry):
        b = s + 16 * k

        @pl.when(b < NVB)
        def _():
            vcopy(b).start()

        return carry

    lax.fori_loop(0, (NVB + 15) // 16, vblock, 0)

    @pl.when(s == 0)
    def _boundary():
        pltpu.sync_copy(eblk_hbm.at[0], bebuf)
        for g in range(2):
            bbuf[pl.ds(16 * g, 16)] = tbl[pl.ds(NVB * 128 + 16 * g, 16)]
        for g in range(2, 8):
            v0 = bebuf[0, pl.ds(16 * g - 32, 16)]
            v1 = bebuf[1, pl.ds(16 * g - 32, 16)]
            g0 = plsc.load_gather(tbl, [v0])
            g1 = plsc.load_gather(tbl, [v1])
            bbuf[pl.ds(16 * g, 16)] = jnp.maximum(g0, g1) + EPS_LUB
        pltpu.sync_copy(bbuf, out_hbm.at[BOUND, col])

    @pl.when(s == 1)
    def _tail():
        pltpu.sync_copy(eblk_hbm.at[NEB - 1], bebuf)
        for g in range(2):
            v0 = bebuf[0, pl.ds(96 + 16 * g, 16)]
            v1 = bebuf[1, pl.ds(96 + 16 * g, 16)]
            g0 = plsc.load_gather(tbl, [v0])
            g1 = plsc.load_gather(tbl, [v1])
            m = jnp.maximum(g0, g1) + EPS_LUB
            for g2 in range(g, 8, 2):
                bbuf[pl.ds(16 * g2, 16)] = m
        pltpu.sync_copy(bbuf, out_hbm.at[LAST_EB, col])

    tile_b0 = FIRST_EB + s * BPT
    nch = BPT // OB - jnp.where(s == 15, 1, 0)

    def in_copy(k, slot):
        ib0 = tile_b0 - FIRST_EB + k * OB
        return pltpu.make_async_copy(
            eblk_hbm.at[pl.ds(ib0, OB + 1)], ebuf.at[slot], semi.at[slot]
        )

    def out_copy(k, slot):
        ob0 = tile_b0 + k * OB
        return pltpu.make_async_copy(
            obuf.at[slot],
            out_hbm.at[pl.ds(ob0, OB), pl.ds(col, 1), :],
            semo.at[slot],
        )

    in_copy(0, 0).start()
    in_copy(1, 1).start()

    def chunk(k, carry):
        slot = lax.rem(k, 3)
        in_copy(k, slot).wait()

        @pl.when(k + 2 < nch)
        def _():
            in_copy(k + 2, lax.rem(k + 2, 3)).start()

        @pl.when(k >= 3)
        def _():
            out_copy(k - 3, slot).wait()

        @plsc.parallel_loop(0, OB, unroll=4)
        def _blocks(t):
            _emit_block(tbl, ebuf.at[slot], obuf.at[slot], t)

        out_copy(k, slot).start()
        return carry

    lax.fori_loop(0, nch, chunk, 0)
    out_copy(nch - 3, lax.rem(nch - 3, 3)).wait()
    out_copy(nch - 2, lax.rem(nch - 2, 3)).wait()
    out_copy(nch - 1, lax.rem(nch - 1, 3)).wait()

    def vdrain(k, carry):
        b = s + 16 * k

        @pl.when(b < NVB)
        def _():
            vcopy(b).wait()

        return carry

    lax.fori_loop(0, (NVB + 15) // 16, vdrain, 0)

    @pl.when(s == 15)
    def _short_tail():
        k = BPT // OB - 1
        ob0 = tile_b0 + k * OB
        ib0 = ob0 - FIRST_EB
        pltpu.sync_copy(eblk_hbm.at[pl.ds(ib0, OB)],
                        ebuf.at[0, pl.ds(0, OB)])

        @plsc.parallel_loop(0, OB - 1, unroll=2)
        def _blocks(t):
            _emit_block(tbl, ebuf.at[0], obuf.at[0], t)
        pltpu.sync_copy(
            obuf.at[0, pl.ds(0, OB - 1)],
            out_hbm.at[pl.ds(ob0, OB - 1), pl.ds(col, 1), :],
        )


@jax.jit
def _run(fcols, eblk):
    mesh = plsc.VectorSubcoreMesh(
        core_axis_name="c", subcore_axis_name="s"
    )
    k = functools.partial(
        pl.kernel,
        mesh=mesh,
        out_type=jax.ShapeDtypeStruct((NOB, 2, 128), jnp.float32),
        scratch_types=[
            pltpu.VMEM((N_NODES,), jnp.float32),
            pltpu.VMEM((3, OB + 1, 2, 128), jnp.int32),
            pltpu.VMEM((3, OB, 1, 128), jnp.float32),
            pltpu.VMEM((2, 128), jnp.int32),
            pltpu.VMEM((128,), jnp.float32),
            pltpu.SemaphoreType.DMA((3,)),
            pltpu.SemaphoreType.DMA((3,)),
            pltpu.SemaphoreType.DMA,
        ],
        compiler_params=pltpu.CompilerParams(
            use_tc_tiling_on_sc=False, needs_layout_passes=False
        ),
    )(_body)
    return k(fcols, eblk)


def kernel(f_v, edges):
    f_v = f_v.astype(jnp.float32)
    fcols = f_v.T
    eblk = (
        edges.astype(jnp.int32)
        .reshape(NEB, 128, 2)
        .transpose(0, 2, 1)
    )
    out_blk = _run(fcols, eblk)
    out = out_blk.transpose(0, 2, 1).reshape(NOB * 128, 2)
    return out[:M_ROWS]
